# Initial kernel scaffold; baseline (speedup 1.0000x reference)
import jax
import jax.numpy as jnp
from jax.experimental import pallas as pl


def kernel(query_layer2, query_layer3, mem_layer2, mem_layer3):
    raise NotImplementedError("write your pallas kernel here")


# speedup vs baseline: 6.5641x; 6.5641x over previous
import functools

import jax
import jax.numpy as jnp
from jax.experimental import pallas as pl

_K_NN = 5
_K_FINAL = 10
_ROW_TILE = 112


def _score_tile_kernel(q_ref, mem_ref, out_ref):
    q = q_ref[...]
    mem = mem_ref[...]
    qsq = jnp.sum(q * q, axis=1, keepdims=True)
    msq = jnp.sum(mem * mem, axis=1)[None, :]
    qm = jax.lax.dot_general(
        q, mem, (((1,), (1,)), ((), ())),
        preferred_element_type=jnp.float32)
    d2 = jnp.maximum(qsq + msq - 2.0 * qm, 0.0)
    kmean = jnp.mean(jnp.exp(-0.5 * d2), axis=1, keepdims=True)
    dist = jnp.sqrt(d2 + 1e-12)

    vals = dist
    acc = jnp.zeros_like(kmean)
    rem = jnp.full_like(kmean, float(_K_NN))
    for _ in range(_K_NN):
        m = jnp.min(vals, axis=1, keepdims=True)
        eq = vals == m
        cnt = jnp.sum(eq.astype(jnp.float32), axis=1, keepdims=True)
        take = jnp.minimum(cnt, rem)
        acc = acc + take * m
        rem = rem - take
        vals = jnp.where(eq, jnp.inf, vals)
    knn = acc / float(_K_NN)
    out_ref[...] = 0.5 * knn + 0.5 * (1.0 - kmean)


def _layer_scores(q_flat, mem):
    n, c = q_flat.shape
    m = mem.shape[0]
    r = _ROW_TILE
    return pl.pallas_call(
        _score_tile_kernel,
        grid=(n // r,),
        in_specs=[
            pl.BlockSpec((r, c), lambda i: (i, 0)),
            pl.BlockSpec((m, c), lambda i: (0, 0)),
        ],
        out_specs=pl.BlockSpec((r, 1), lambda i: (i, 0)),
        out_shape=jax.ShapeDtypeStruct((n, 1), jnp.float32),
    )(q_flat, mem)


def _final_kernel(c2_ref, c3_ref, map_ref, score_ref):
    cm = 0.5 * (c2_ref[...] + c3_ref[...])
    map_ref[...] = cm
    vals = cm
    acc = jnp.zeros((cm.shape[0], 1), jnp.float32)
    rem = jnp.full((cm.shape[0], 1), float(_K_FINAL), jnp.float32)
    for _ in range(_K_FINAL):
        m = jnp.max(vals, axis=1, keepdims=True)
        eq = vals == m
        cnt = jnp.sum(eq.astype(jnp.float32), axis=1, keepdims=True)
        take = jnp.minimum(cnt, rem)
        acc = acc + take * m
        rem = rem - take
        vals = jnp.where(eq, -jnp.inf, vals)
    score_ref[...] = acc / float(_K_FINAL)


def kernel(query_layer2, query_layer3, mem_layer2, mem_layer3):
    b, c2, h, w = query_layer2.shape
    c3 = query_layer3.shape[1]
    q2 = jnp.transpose(query_layer2, (0, 2, 3, 1)).reshape(-1, c2)
    q3 = jnp.transpose(query_layer3, (0, 2, 3, 1)).reshape(-1, c3)

    comb2 = _layer_scores(q2, mem_layer2)
    comb3 = _layer_scores(q3, mem_layer3)

    map2 = comb2.reshape(b, h, w)
    map3 = comb3.reshape(b, h, w)

    cm, scores = pl.pallas_call(
        _final_kernel,
        out_shape=(
            jax.ShapeDtypeStruct((b, h * w), jnp.float32),
            jax.ShapeDtypeStruct((b, 1), jnp.float32),
        ),
    )(comb2.reshape(b, h * w), comb3.reshape(b, h * w))

    return (scores[:, 0], cm.reshape(b, h, w), map2, map3)


# speedup vs baseline: 8.1962x; 1.2486x over previous
import functools

import jax
import jax.numpy as jnp
from jax.experimental import pallas as pl
from jax.experimental.pallas import tpu as pltpu

_K_NN = 5
_K_FINAL = 10
_ROW_TILE = 112


def _score_tile_kernel(q_ref, mem_ref, out_ref, msq_ref):
    @pl.when(pl.program_id(0) == 0)
    def _():
        mem0 = mem_ref[...]
        msq_ref[...] = jnp.sum(mem0 * mem0, axis=1)[None, :]

    q = q_ref[...]
    qsq = jnp.sum(q * q, axis=1, keepdims=True)
    qm = jax.lax.dot_general(
        q, mem_ref[...], (((1,), (1,)), ((), ())),
        preferred_element_type=jnp.float32)
    d2 = jnp.maximum(qsq + msq_ref[...] - 2.0 * qm, 0.0)
    kmean = jnp.mean(jnp.exp(-0.5 * d2), axis=1, keepdims=True)

    vals = d2
    acc = jnp.zeros_like(kmean)
    rem = jnp.full_like(kmean, float(_K_NN))
    for _ in range(_K_NN):
        m = jnp.min(vals, axis=1, keepdims=True)
        eq = vals == m
        cnt = jnp.sum(eq.astype(jnp.float32), axis=1, keepdims=True)
        take = jnp.minimum(cnt, rem)
        acc = acc + take * jnp.sqrt(m + 1e-12)
        rem = rem - take
        vals = jnp.where(eq, jnp.inf, vals)
    knn = acc / float(_K_NN)
    out_ref[...] = 0.5 * knn + 0.5 * (1.0 - kmean)


def _layer_scores(q_flat, mem):
    n, c = q_flat.shape
    m = mem.shape[0]
    r = _ROW_TILE
    return pl.pallas_call(
        _score_tile_kernel,
        grid=(n // r,),
        in_specs=[
            pl.BlockSpec((r, c), lambda i: (i, 0)),
            pl.BlockSpec((m, c), lambda i: (0, 0)),
        ],
        out_specs=pl.BlockSpec((r, 1), lambda i: (i, 0)),
        out_shape=jax.ShapeDtypeStruct((n, 1), jnp.float32),
        scratch_shapes=[pltpu.VMEM((1, m), jnp.float32)],
    )(q_flat, mem)


def _final_kernel(c2_ref, c3_ref, map_ref, score_ref):
    cm = 0.5 * (c2_ref[...] + c3_ref[...])
    map_ref[...] = cm
    vals = cm
    acc = jnp.zeros((cm.shape[0], 1), jnp.float32)
    rem = jnp.full((cm.shape[0], 1), float(_K_FINAL), jnp.float32)
    for _ in range(_K_FINAL):
        m = jnp.max(vals, axis=1, keepdims=True)
        eq = vals == m
        cnt = jnp.sum(eq.astype(jnp.float32), axis=1, keepdims=True)
        take = jnp.minimum(cnt, rem)
        acc = acc + take * m
        rem = rem - take
        vals = jnp.where(eq, -jnp.inf, vals)
    score_ref[...] = acc / float(_K_FINAL)


def kernel(query_layer2, query_layer3, mem_layer2, mem_layer3):
    b, c2, h, w = query_layer2.shape
    c3 = query_layer3.shape[1]
    q2 = jnp.transpose(query_layer2, (0, 2, 3, 1)).reshape(-1, c2)
    q3 = jnp.transpose(query_layer3, (0, 2, 3, 1)).reshape(-1, c3)

    comb2 = _layer_scores(q2, mem_layer2)
    comb3 = _layer_scores(q3, mem_layer3)

    map2 = comb2.reshape(b, h, w)
    map3 = comb3.reshape(b, h, w)

    cm, scores = pl.pallas_call(
        _final_kernel,
        out_shape=(
            jax.ShapeDtypeStruct((b, h * w), jnp.float32),
            jax.ShapeDtypeStruct((b, 1), jnp.float32),
        ),
    )(comb2.reshape(b, h * w), comb3.reshape(b, h * w))

    return (scores[:, 0], cm.reshape(b, h, w), map2, map3)


# speedup vs baseline: 12.2173x; 1.4906x over previous
import functools

import jax
import jax.numpy as jnp
from jax.experimental import pallas as pl
from jax.experimental.pallas import tpu as pltpu

_K_NN = 5
_K_FINAL = 10
_ROW_TILE = 112


def _score_tile_kernel(q_ref, mem_ref, out_ref, msq_ref):
    @pl.when(pl.program_id(0) == 0)
    def _():
        mem0 = mem_ref[...]
        msq_ref[...] = jnp.sum(mem0 * mem0, axis=1)[None, :]

    q = q_ref[...]
    qsq = jnp.sum(q * q, axis=1, keepdims=True)
    qm = jax.lax.dot_general(
        q, mem_ref[...], (((1,), (1,)), ((), ())),
        preferred_element_type=jnp.float32)
    d2 = jnp.maximum(qsq + msq_ref[...] - 2.0 * qm, 0.0)
    kmean = jnp.mean(jnp.exp(-0.5 * d2), axis=1, keepdims=True)

    r, m_total = d2.shape
    w = 128
    tops = [jnp.full((r, w), jnp.inf, jnp.float32) for _ in range(_K_NN)]
    for s in range(m_total // w):
        x = d2[:, s * w:(s + 1) * w]
        for j in range(_K_NN):
            lo = jnp.minimum(tops[j], x)
            x = jnp.maximum(tops[j], x)
            tops[j] = lo
    vals = jnp.concatenate(tops, axis=1)
    acc = jnp.zeros_like(kmean)
    rem = jnp.full_like(kmean, float(_K_NN))
    for _ in range(_K_NN):
        m = jnp.min(vals, axis=1, keepdims=True)
        eq = vals == m
        cnt = jnp.sum(eq.astype(jnp.float32), axis=1, keepdims=True)
        take = jnp.minimum(cnt, rem)
        acc = acc + take * jnp.sqrt(m + 1e-12)
        rem = rem - take
        vals = jnp.where(eq, jnp.inf, vals)
    knn = acc / float(_K_NN)
    out_ref[...] = 0.5 * knn + 0.5 * (1.0 - kmean)


def _layer_scores(q_flat, mem):
    n, c = q_flat.shape
    m = mem.shape[0]
    r = _ROW_TILE
    return pl.pallas_call(
        _score_tile_kernel,
        grid=(n // r,),
        in_specs=[
            pl.BlockSpec((r, c), lambda i: (i, 0)),
            pl.BlockSpec((m, c), lambda i: (0, 0)),
        ],
        out_specs=pl.BlockSpec((r, 1), lambda i: (i, 0)),
        out_shape=jax.ShapeDtypeStruct((n, 1), jnp.float32),
        scratch_shapes=[pltpu.VMEM((1, m), jnp.float32)],
    )(q_flat, mem)


def _final_kernel(c2_ref, c3_ref, map_ref, score_ref):
    cm = 0.5 * (c2_ref[...] + c3_ref[...])
    map_ref[...] = cm
    vals = cm
    acc = jnp.zeros((cm.shape[0], 1), jnp.float32)
    rem = jnp.full((cm.shape[0], 1), float(_K_FINAL), jnp.float32)
    for _ in range(_K_FINAL):
        m = jnp.max(vals, axis=1, keepdims=True)
        eq = vals == m
        cnt = jnp.sum(eq.astype(jnp.float32), axis=1, keepdims=True)
        take = jnp.minimum(cnt, rem)
        acc = acc + take * m
        rem = rem - take
        vals = jnp.where(eq, -jnp.inf, vals)
    score_ref[...] = acc / float(_K_FINAL)


def kernel(query_layer2, query_layer3, mem_layer2, mem_layer3):
    b, c2, h, w = query_layer2.shape
    c3 = query_layer3.shape[1]
    q2 = jnp.transpose(query_layer2, (0, 2, 3, 1)).reshape(-1, c2)
    q3 = jnp.transpose(query_layer3, (0, 2, 3, 1)).reshape(-1, c3)

    comb2 = _layer_scores(q2, mem_layer2)
    comb3 = _layer_scores(q3, mem_layer3)

    map2 = comb2.reshape(b, h, w)
    map3 = comb3.reshape(b, h, w)

    cm, scores = pl.pallas_call(
        _final_kernel,
        out_shape=(
            jax.ShapeDtypeStruct((b, h * w), jnp.float32),
            jax.ShapeDtypeStruct((b, 1), jnp.float32),
        ),
    )(comb2.reshape(b, h * w), comb3.reshape(b, h * w))

    return (scores[:, 0], cm.reshape(b, h, w), map2, map3)
